# baseline (speedup 1.0000x reference)
import functools

import numpy as np

import jax
import jax.numpy as jnp
from jax import lax
from jax.experimental import pallas as pl
from jax.experimental.pallas import tpu as pltpu
from jax.experimental.pallas import tpu_sc as plsc

N = 10000
E = 320000
D_IN = 128
HID = 64
NUM_LAYERS = 12
R = 3
NG = 256
HEADS = 16
GREP = 416
CAT = HID * (NUM_LAYERS + 1)
MLP_H = 256

_NC = 2
_NS = 16
_NW = _NC * _NS
_EPW = E // _NW
_K = 128
_NFULL = _EPW // _K
_REM = _EPW - _NFULL * _K
_RPT_STEP = 624
_RPT_TAIL = N - _NS * _RPT_STEP
_CPT_STEP = 1872
_CPT_TAIL = R * N - _NS * _CPT_STEP
_ZROWS = 48

_BN = 1000
_GN = N // _BN
_WCOLS = 3 * HID + R * HID + R * 2 * HID

_PERM32 = np.stack([np.arange(16), np.arange(16) + 16], axis=1).reshape(32)
_COLPERM = np.arange(_WCOLS)
for _blk in range(3 * HID, 3 * HID + R * HID, 32):
    _COLPERM[_blk:_blk + 32] = _blk + _PERM32


def _leaky(v):
    return jnp.maximum(v, 0.01 * v)


def _sc_count(if3, if_r):
    mesh = plsc.VectorSubcoreMesh(
        core_axis_name="c", subcore_axis_name="s",
        num_cores=_NC, num_subcores=_NS)

    @functools.partial(
        pl.kernel,
        out_type=jax.ShapeDtypeStruct((_NC * R * N, 16), jnp.float32),
        mesh=mesh,
        compiler_params=pltpu.CompilerParams(use_tc_tiling_on_sc=False),
        scratch_types=[
            pltpu.VMEM((_NFULL, _K), jnp.int32),
            pltpu.VMEM((_K, 16), jnp.float32),
            pltpu.VMEM((_REM,), jnp.int32),
            pltpu.VMEM((_REM, 16), jnp.float32),
            pltpu.VMEM((_CPT_STEP, 16), jnp.float32),
            pltpu.VMEM_SHARED((R * N, 16), jnp.float32),
        ],
    )
    def k(if_h, ifr_h, out_h, if2d, ones_v, iv2, ones2_v, zb, acc):
        c = lax.axis_index("c")
        s = lax.axis_index("s")
        wid = s * _NC + c
        one16 = jnp.full((16,), 1.0, jnp.float32)
        zero16 = jnp.zeros((16,), jnp.float32)

        def fill_z(i, _):
            zb[i, :] = zero16
            return 0
        lax.fori_loop(0, _CPT_STEP, fill_z, 0)

        def fill_o(i, _):
            ones_v[i, :] = one16
            return 0
        lax.fori_loop(0, _K, fill_o, 0)
        for i in range(_REM):
            ones2_v[i, :] = one16

        pltpu.sync_copy(zb, acc.at[pl.ds(s * _CPT_STEP, _CPT_STEP)])

        @pl.when(s == _NS - 1)
        def _():
            pltpu.sync_copy(zb.at[pl.ds(0, _CPT_TAIL)],
                            acc.at[pl.ds(_NS * _CPT_STEP, _CPT_TAIL)])

        pltpu.sync_copy(if_h.at[wid], if2d)
        pltpu.sync_copy(ifr_h.at[wid], iv2)
        plsc.subcore_barrier()

        def chunk(j, _):
            pltpu.sync_copy(ones_v, acc.at[if2d.at[j]], add=True)
            return 0
        lax.fori_loop(0, _NFULL, chunk, 0)
        pltpu.sync_copy(ones2_v, acc.at[iv2], add=True)

        plsc.subcore_barrier()
        dsto = pl.multiple_of(c * (R * N) + s * _CPT_STEP, 8)
        pltpu.sync_copy(acc.at[pl.ds(s * _CPT_STEP, _CPT_STEP)],
                        out_h.at[pl.ds(dsto, _CPT_STEP)])

        @pl.when(s == _NS - 1)
        def _():
            to = pl.multiple_of(c * (R * N) + _NS * _CPT_STEP, 8)
            pltpu.sync_copy(acc.at[pl.ds(_NS * _CPT_STEP, _CPT_TAIL)],
                            out_h.at[pl.ds(to, _CPT_TAIL)])

    return k(if3, if_r)


def _sc_edge(m_tbl, f_tbl, im3, if3, dv3, im_r, if_r, dv_r):
    mesh = plsc.VectorSubcoreMesh(
        core_axis_name="c", subcore_axis_name="s",
        num_cores=_NC, num_subcores=_NS)

    @functools.partial(
        pl.kernel,
        out_type=jax.ShapeDtypeStruct((_NC * N, HID), jnp.float32),
        mesh=mesh,
        compiler_params=pltpu.CompilerParams(
            use_tc_tiling_on_sc=False, needs_layout_passes=False),
        scratch_types=[
            pltpu.VMEM((_NFULL, _K), jnp.int32),
            pltpu.VMEM((_NFULL, _K), jnp.int32),
            pltpu.VMEM((_NFULL, _K), jnp.int32),
            pltpu.VMEM((_K, HID), jnp.bfloat16),
            pltpu.VMEM((_K, 2 * HID), jnp.float32),
            pltpu.VMEM((_K, HID), jnp.bfloat16),
            pltpu.VMEM((_K, 2 * HID), jnp.float32),
            pltpu.VMEM((_K, HID), jnp.float32),
            pltpu.VMEM((_REM,), jnp.int32),
            pltpu.VMEM((_REM,), jnp.int32),
            pltpu.VMEM((_REM,), jnp.int32),
            pltpu.VMEM((_ZROWS, HID), jnp.float32),
            pltpu.VMEM_SHARED((N, HID), jnp.float32),
            pltpu.SemaphoreType.DMA,
            pltpu.SemaphoreType.DMA,
            pltpu.SemaphoreType.DMA,
            pltpu.SemaphoreType.DMA,
        ],
    )
    def k(mtbl, ftbl, im_h, if_h, dv_h, imr_h, ifr_h, dvr_h, out_h,
          im2d, if2d, dv2d, mb0, fb0, mb1, fb1, mg,
          imr, ifr, dvr,
          zb, acc, sm0, sf0, sm1, sf1):
        c = lax.axis_index("c")
        s = lax.axis_index("s")
        wid = s * _NC + c
        zero16 = jnp.zeros((16,), jnp.float32)

        def fill_z(i, _):
            for g in range(HID // 16):
                zb[i, pl.ds(g * 16, 16)] = zero16
            return 0
        lax.fori_loop(0, _ZROWS, fill_z, 0)
        for q in range(_RPT_STEP // _ZROWS):
            pltpu.sync_copy(
                zb, acc.at[pl.ds(s * _RPT_STEP + q * _ZROWS, _ZROWS)])

        @pl.when(s == _NS - 1)
        def _():
            pltpu.sync_copy(zb.at[pl.ds(0, _RPT_TAIL)],
                            acc.at[pl.ds(_NS * _RPT_STEP, _RPT_TAIL)])

        pltpu.sync_copy(im_h.at[wid], im2d)
        pltpu.sync_copy(if_h.at[wid], if2d)
        pltpu.sync_copy(dv_h.at[wid], dv2d)
        pltpu.sync_copy(imr_h.at[wid], imr)
        pltpu.sync_copy(ifr_h.at[wid], ifr)
        pltpu.sync_copy(dvr_h.at[wid], dvr)
        plsc.subcore_barrier()

        bufs = ((mb0, fb0, sm0, sf0), (mb1, fb1, sm1, sf1))

        def g_issue(j, slot):
            mb, fb, smx, sfx = bufs[slot]
            pltpu.async_copy(mtbl.at[im2d.at[j]], mb, smx)
            pltpu.async_copy(ftbl.at[if2d.at[j]], fb, sfx)

        def g_wait(j, slot):
            mb, fb, smx, sfx = bufs[slot]
            pltpu.make_async_copy(mtbl.at[im2d.at[j]], mb, smx).wait()
            pltpu.make_async_copy(ftbl.at[if2d.at[j]], fb, sfx).wait()

        def _unp(x):
            return plsc.unpack(x, format=plsc.PackFormat.INTERLEAVED,
                               preferred_element_type=jnp.float32)

        def compute(mb, fb, nk):
            def edge(kk, _):
                for b in range(HID // 32):
                    ma, mc = _unp(mb[kk, pl.ds(b * 32, 32)])
                    for half, mv in ((0, ma), (1, mc)):
                        g = 2 * b + half
                        bv = fb[kk, pl.ds(g * 16, 16)]
                        gv = fb[kk, pl.ds(HID + g * 16, 16)]
                        v = gv * mv + bv
                        mg[kk, pl.ds(g * 16, 16)] = jnp.maximum(v, 0.01 * v)
                return 0
            lax.fori_loop(0, nk, edge, 0)

        g_issue(0, 0)
        g_issue(1, 1)

        def chunk2(jj, _):
            j0 = jj * 2
            for slot in range(2):
                j = j0 + slot
                mb, fb, _, _ = bufs[slot]
                g_wait(j, slot)
                compute(mb, fb, _K)

                @pl.when(j + 2 < _NFULL)
                def _():
                    g_issue(j + 2, slot)
                pltpu.sync_copy(mg, acc.at[dv2d.at[j]], add=True)
            return 0
        lax.fori_loop(0, _NFULL // 2, chunk2, 0)

        cm = pltpu.async_copy(mtbl.at[imr], mb0.at[pl.ds(0, _REM)], sm0)
        cf = pltpu.async_copy(ftbl.at[ifr], fb0.at[pl.ds(0, _REM)], sf0)
        cm.wait()
        cf.wait()
        compute(mb0, fb0, _REM)
        pltpu.sync_copy(mg.at[pl.ds(0, _REM)], acc.at[dvr], add=True)

        plsc.subcore_barrier()
        dsto = pl.multiple_of(c * N + s * _RPT_STEP, 8)
        pltpu.sync_copy(acc.at[pl.ds(s * _RPT_STEP, _RPT_STEP)],
                        out_h.at[pl.ds(dsto, _RPT_STEP)])

        @pl.when(s == _NS - 1)
        def _():
            to = pl.multiple_of(c * N + _NS * _RPT_STEP, 8)
            pltpu.sync_copy(acc.at[pl.ds(_NS * _RPT_STEP, _RPT_TAIL)],
                            out_h.at[pl.ds(to, _RPT_TAIL)])

    return k(m_tbl, f_tbl, im3, if3, dv3, im_r, if_r, dv_r)


def _rep(shape):
    nd = len(shape)
    return pl.BlockSpec(shape, lambda i: (0,) * nd)


def _tc_layer0(x, wcat, bcat, wtbl):
    def body(x_ref, wcat_ref, bcat_ref, wtbl_ref, sk_ref, m_ref, f_ref):
        hn = x_ref[...]
        big = jnp.dot(hn, wcat_ref[...],
                      preferred_element_type=jnp.float32) + bcat_ref[...]
        v = big[:, 2 * HID:3 * HID] * big[:, 0:HID] + big[:, HID:2 * HID]
        sk_ref[...] = _leaky(v)
        for r in range(R):
            m_ref[r] = big[:, 3 * HID + r * HID:
                           3 * HID + (r + 1) * HID].astype(jnp.bfloat16)
            fv = big[:, 3 * HID + R * HID + r * 2 * HID:
                     3 * HID + R * HID + (r + 1) * 2 * HID]
            f_ref[r] = fv * wtbl_ref[:, r:r + 1]

    return pl.pallas_call(
        body,
        grid=(_GN,),
        in_specs=[
            pl.BlockSpec((_BN, D_IN), lambda i: (i, 0)),
            _rep((D_IN, _WCOLS)),
            _rep((1, _WCOLS)),
            pl.BlockSpec((_BN, R), lambda i: (i, 0)),
        ],
        out_specs=[
            pl.BlockSpec((_BN, HID), lambda i: (i, 0)),
            pl.BlockSpec((R, _BN, HID), lambda i: (0, i, 0)),
            pl.BlockSpec((R, _BN, 2 * HID), lambda i: (0, i, 0)),
        ],
        out_shape=[
            jax.ShapeDtypeStruct((N, HID), jnp.float32),
            jax.ShapeDtypeStruct((R, N, HID), jnp.bfloat16),
            jax.ShapeDtypeStruct((R, N, 2 * HID), jnp.float32),
        ],
    )(x, wcat, bcat, wtbl)


def _tc_layer_hid(skip, p0, p1, lng, lnb, wcat, bcat, wtbl):
    def body(skip_ref, p0_ref, p1_ref, lng_ref, lnb_ref, wcat_ref, bcat_ref,
             wtbl_ref, h_ref, sk_ref, m_ref, f_ref):
        h = skip_ref[...] + p0_ref[...] + p1_ref[...]
        h_ref[...] = h
        mu = jnp.mean(h, axis=1, keepdims=True)
        d = h - mu
        var = jnp.mean(d * d, axis=1, keepdims=True)
        hn = d * lax.rsqrt(var + 1e-5) * lng_ref[...] + lnb_ref[...]
        big = jnp.dot(hn, wcat_ref[...],
                      preferred_element_type=jnp.float32) + bcat_ref[...]
        v = big[:, 2 * HID:3 * HID] * big[:, 0:HID] + big[:, HID:2 * HID]
        sk_ref[...] = _leaky(v)
        for r in range(R):
            m_ref[r] = big[:, 3 * HID + r * HID:
                           3 * HID + (r + 1) * HID].astype(jnp.bfloat16)
            fv = big[:, 3 * HID + R * HID + r * 2 * HID:
                     3 * HID + R * HID + (r + 1) * 2 * HID]
            f_ref[r] = fv * wtbl_ref[:, r:r + 1]

    return pl.pallas_call(
        body,
        grid=(_GN,),
        in_specs=[
            pl.BlockSpec((_BN, HID), lambda i: (i, 0)),
            pl.BlockSpec((_BN, HID), lambda i: (i, 0)),
            pl.BlockSpec((_BN, HID), lambda i: (i, 0)),
            _rep((1, HID)),
            _rep((1, HID)),
            _rep((HID, _WCOLS)),
            _rep((1, _WCOLS)),
            pl.BlockSpec((_BN, R), lambda i: (i, 0)),
        ],
        out_specs=[
            pl.BlockSpec((_BN, HID), lambda i: (i, 0)),
            pl.BlockSpec((_BN, HID), lambda i: (i, 0)),
            pl.BlockSpec((R, _BN, HID), lambda i: (0, i, 0)),
            pl.BlockSpec((R, _BN, 2 * HID), lambda i: (0, i, 0)),
        ],
        out_shape=[
            jax.ShapeDtypeStruct((N, HID), jnp.float32),
            jax.ShapeDtypeStruct((N, HID), jnp.float32),
            jax.ShapeDtypeStruct((R, N, HID), jnp.bfloat16),
            jax.ShapeDtypeStruct((R, N, 2 * HID), jnp.float32),
        ],
    )(skip, p0, p1, lng, lnb, wcat, bcat, wtbl)


def _tc_pool_a(precat, skip, p0, p1, batch3, sw1, sb1, sw2, sb2, sw3, sb3,
               tw1, tb1, tw2, tb2, tw3, tb3):
    def body(pc_ref, skip_ref, p0_ref, p1_ref, b_ref,
             sw1_ref, sb1_ref, sw2_ref, sb2_ref, sw3_ref, sb3_ref,
             tw1_ref, tb1_ref, tw2_ref, tb2_ref, tw3_ref, tb3_ref,
             cat_ref, sc_ref, tr_ref, smax_ref):
        h12 = skip_ref[...] + p0_ref[...] + p1_ref[...]
        cat = jnp.concatenate([pc_ref[...], h12], axis=1)
        cat_ref[...] = cat
        s1 = _leaky(jnp.dot(cat, sw1_ref[...],
                            preferred_element_type=jnp.float32) + sb1_ref[...])
        s2 = _leaky(jnp.dot(s1, sw2_ref[...],
                            preferred_element_type=jnp.float32) + sb2_ref[...])
        sc = jnp.dot(s2, sw3_ref[...],
                     preferred_element_type=jnp.float32) + sb3_ref[...]
        sc_ref[...] = sc
        t1 = _leaky(jnp.dot(cat, tw1_ref[...],
                            preferred_element_type=jnp.float32) + tb1_ref[...])
        t2 = _leaky(jnp.dot(t1, tw2_ref[...],
                            preferred_element_type=jnp.float32) + tb2_ref[...])
        tr = _leaky(jnp.dot(t2, tw3_ref[...],
                            preferred_element_type=jnp.float32) + tb3_ref[...])
        tr_ref[...] = tr

        b = b_ref[0, 0, :]
        ids = lax.broadcasted_iota(jnp.int32, (_BN, NG), 1)
        mask = b[:, None] == ids

        @pl.when(pl.program_id(0) == 0)
        def _():
            smax_ref[...] = jnp.full((NG, HEADS), -1e30, jnp.float32)

        cols = []
        for h in range(HEADS):
            cur = jnp.where(mask, sc[:, h:h + 1], -1e30)
            cols.append(jnp.max(cur, axis=0, keepdims=True))
        upd = jnp.concatenate(cols, axis=0).T
        smax_ref[...] = jnp.maximum(smax_ref[...], upd)

    return pl.pallas_call(
        body,
        grid=(_GN,),
        in_specs=[
            pl.BlockSpec((_BN, CAT - HID), lambda i: (i, 0)),
            pl.BlockSpec((_BN, HID), lambda i: (i, 0)),
            pl.BlockSpec((_BN, HID), lambda i: (i, 0)),
            pl.BlockSpec((_BN, HID), lambda i: (i, 0)),
            pl.BlockSpec((1, 1, _BN), lambda i: (i, 0, 0)),
            _rep((CAT, MLP_H)), _rep((1, MLP_H)),
            _rep((MLP_H, MLP_H)), _rep((1, MLP_H)),
            _rep((MLP_H, HEADS)), _rep((1, HEADS)),
            _rep((CAT, MLP_H)), _rep((1, MLP_H)),
            _rep((MLP_H, MLP_H)), _rep((1, MLP_H)),
            _rep((MLP_H, GREP)), _rep((1, GREP)),
        ],
        out_specs=[
            pl.BlockSpec((_BN, CAT), lambda i: (i, 0)),
            pl.BlockSpec((_BN, HEADS), lambda i: (i, 0)),
            pl.BlockSpec((_BN, GREP), lambda i: (i, 0)),
            pl.BlockSpec((NG, HEADS), lambda i: (0, 0)),
        ],
        out_shape=[
            jax.ShapeDtypeStruct((N, CAT), jnp.float32),
            jax.ShapeDtypeStruct((N, HEADS), jnp.float32),
            jax.ShapeDtypeStruct((N, GREP), jnp.float32),
            jax.ShapeDtypeStruct((NG, HEADS), jnp.float32),
        ],
    )(precat, skip, p0, p1, batch3, sw1, sb1, sw2, sb2, sw3, sb3,
      tw1, tb1, tw2, tb2, tw3, tb3)


def _tc_pool_b(scores, batch3, smax):
    def body(sc_ref, b_ref, smax_ref, ex_ref, ssum_ref):
        b = b_ref[0, 0, :]
        ids = lax.broadcasted_iota(jnp.int32, (_BN, NG), 1)
        maskf = (b[:, None] == ids).astype(jnp.float32)
        idsT = lax.broadcasted_iota(jnp.int32, (NG, _BN), 0)
        maskTf = (b[None, :] == idsT).astype(jnp.float32)
        smax_node = jnp.dot(maskf, smax_ref[...],
                            preferred_element_type=jnp.float32)
        ex = jnp.exp(sc_ref[...] - smax_node)
        ex_ref[...] = ex

        @pl.when(pl.program_id(0) == 0)
        def _():
            ssum_ref[...] = jnp.zeros((NG, HEADS), jnp.float32)

        ssum_ref[...] += jnp.dot(maskTf, ex,
                                 preferred_element_type=jnp.float32)

    return pl.pallas_call(
        body,
        grid=(_GN,),
        in_specs=[
            pl.BlockSpec((_BN, HEADS), lambda i: (i, 0)),
            pl.BlockSpec((1, 1, _BN), lambda i: (i, 0, 0)),
            _rep((NG, HEADS)),
        ],
        out_specs=[
            pl.BlockSpec((_BN, HEADS), lambda i: (i, 0)),
            pl.BlockSpec((NG, HEADS), lambda i: (0, 0)),
        ],
        out_shape=[
            jax.ShapeDtypeStruct((N, HEADS), jnp.float32),
            jax.ShapeDtypeStruct((NG, HEADS), jnp.float32),
        ],
    )(scores, batch3, smax)


def _tc_pool_c(ex, trans, batch3, ssum):
    def body(ex_ref, tr_ref, b_ref, ssum_ref, gr_ref):
        b = b_ref[0, 0, :]
        ids = lax.broadcasted_iota(jnp.int32, (_BN, NG), 1)
        maskf = (b[:, None] == ids).astype(jnp.float32)
        idsT = lax.broadcasted_iota(jnp.int32, (NG, _BN), 0)
        maskTf = (b[None, :] == idsT).astype(jnp.float32)
        ssum_node = jnp.dot(maskf, ssum_ref[...],
                            preferred_element_type=jnp.float32)
        w = ex_ref[...] / jnp.maximum(ssum_node, 1e-16)
        hsel = (lax.broadcasted_iota(jnp.int32, (HEADS, GREP), 1)
                // (GREP // HEADS)
                == lax.broadcasted_iota(jnp.int32, (HEADS, GREP), 0)
                ).astype(jnp.float32)
        wrep = jnp.dot(w, hsel, preferred_element_type=jnp.float32)
        weighted = wrep * tr_ref[...]

        @pl.when(pl.program_id(0) == 0)
        def _():
            gr_ref[...] = jnp.zeros((NG, GREP), jnp.float32)

        gr_ref[...] += jnp.dot(maskTf, weighted,
                               preferred_element_type=jnp.float32)

    return pl.pallas_call(
        body,
        grid=(_GN,),
        in_specs=[
            pl.BlockSpec((_BN, HEADS), lambda i: (i, 0)),
            pl.BlockSpec((_BN, GREP), lambda i: (i, 0)),
            pl.BlockSpec((1, 1, _BN), lambda i: (i, 0, 0)),
            _rep((NG, HEADS)),
        ],
        out_specs=[pl.BlockSpec((NG, GREP), lambda i: (0, 0))],
        out_shape=[jax.ShapeDtypeStruct((NG, GREP), jnp.float32)],
    )(ex, trans, batch3, ssum)[0]


def _cat_weights(lin_skip, film_skip_w, film_skip_b, lins, films_w, films_b):
    wcat = jnp.concatenate(
        [lin_skip, film_skip_w]
        + [lins[r] for r in range(R)]
        + [films_w[r] for r in range(R)], axis=1)
    bcat = jnp.concatenate(
        [jnp.zeros((HID,), jnp.float32), film_skip_b,
         jnp.zeros((R * HID,), jnp.float32), films_b.reshape(-1)])[None]
    return wcat[:, _COLPERM], bcat[:, _COLPERM]


def kernel(x, edge_index, edge_type, batch, params):
    src = edge_index[0]
    dst = edge_index[1]
    et = edge_type.astype(jnp.int32)
    idx_m = et * N + src
    idx_f = et * N + dst

    def _split(a):
        a = a.reshape(_NW, _EPW)
        return (a[:, :_NFULL * _K].reshape(_NW, _NFULL, _K),
                a[:, _NFULL * _K:])

    im3, im_r = _split(idx_m)
    if3, if_r = _split(idx_f)
    dv3, dv_r = _split(dst)

    cnt2 = _sc_count(if3, if_r)
    cnt = cnt2.reshape(_NC, R * N, 16)[:, :, 0].sum(axis=0)
    wtbl = (1.0 / jnp.maximum(cnt, 1.0)).reshape(R, N).T

    p0 = params['l0']
    ph = params['hid']
    pp = params['pool']

    wcat0, bcat0 = _cat_weights(
        p0['lin_skip'], p0['film_skip_w'], p0['film_skip_b'],
        p0['lins'], p0['films_w'], p0['films_b'])
    skip, m_all, f_all = _tc_layer0(x, wcat0, bcat0, wtbl)
    part = _sc_edge(m_all.reshape(R * N, HID), f_all.reshape(R * N, 2 * HID),
                    im3, if3, dv3, im_r, if_r, dv_r).reshape(_NC, N, HID)

    hs = []
    for l in range(NUM_LAYERS):
        wcat, bcat = _cat_weights(
            ph['lin_skip'][l], ph['film_skip_w'][l], ph['film_skip_b'][l],
            ph['lins'][l], ph['films_w'][l], ph['films_b'][l])
        h_prev, skip, m_all, f_all = _tc_layer_hid(
            skip, part[0], part[1], ph['ln_g'][l][None], ph['ln_b'][l][None],
            wcat, bcat, wtbl)
        hs.append(h_prev)
        part = _sc_edge(
            m_all.reshape(R * N, HID), f_all.reshape(R * N, 2 * HID),
            im3, if3, dv3, im_r, if_r, dv_r).reshape(_NC, N, HID)

    precat = jnp.concatenate(hs, axis=1)
    batch3 = batch.astype(jnp.int32).reshape(_GN, 1, _BN)
    cat, scores, trans, smax = _tc_pool_a(
        precat, skip, part[0], part[1], batch3,
        pp['score_w1'], pp['score_b1'][None], pp['score_w2'],
        pp['score_b2'][None], pp['score_w3'], pp['score_b3'][None],
        pp['trans_w1'], pp['trans_b1'][None], pp['trans_w2'],
        pp['trans_b2'][None], pp['trans_w3'], pp['trans_b3'][None])
    ex, ssum = _tc_pool_b(scores, batch3, smax)
    graph_reprs = _tc_pool_c(ex, trans, batch3, ssum)
    return graph_reprs, cat


# speedup vs baseline: 1.0569x; 1.0569x over previous
import functools

import jax
import jax.numpy as jnp
from jax import lax
from jax.experimental import pallas as pl
from jax.experimental.pallas import tpu as pltpu
from jax.experimental.pallas import tpu_sc as plsc

N = 10000
E = 320000
D_IN = 128
HID = 64
NUM_LAYERS = 12
R = 3
NG = 256
HEADS = 16
GREP = 416
CAT = HID * (NUM_LAYERS + 1)
MLP_H = 256

_NC = 2
_NS = 16
_NW = _NC * _NS
_EPW = E // _NW
_K = 128
_NFULL = _EPW // _K
_REM = _EPW - _NFULL * _K
_RPT_STEP = 624
_RPT_TAIL = N - _NS * _RPT_STEP
_CPT_STEP = 1872
_CPT_TAIL = R * N - _NS * _CPT_STEP
_PH0 = 40
_PH1 = _NFULL - _PH0

_BN = 1000
_GN = N // _BN
_WCOLS = 3 * HID + R * HID + R * 2 * HID

def _leaky(v):
    return jnp.maximum(v, 0.01 * v)


def _sc_count(if3, if_r):
    mesh = plsc.VectorSubcoreMesh(
        core_axis_name="c", subcore_axis_name="s",
        num_cores=_NC, num_subcores=_NS)

    @functools.partial(
        pl.kernel,
        out_type=jax.ShapeDtypeStruct((_NC * R * N, 16), jnp.float32),
        mesh=mesh,
        compiler_params=pltpu.CompilerParams(use_tc_tiling_on_sc=False),
        scratch_types=[
            pltpu.VMEM((_NFULL, _K), jnp.int32),
            pltpu.VMEM((_K, 16), jnp.float32),
            pltpu.VMEM((_REM,), jnp.int32),
            pltpu.VMEM((_REM, 16), jnp.float32),
            pltpu.VMEM((_CPT_STEP, 16), jnp.float32),
            pltpu.VMEM_SHARED((R * N, 16), jnp.float32),
            pltpu.SemaphoreType.DMA,
        ],
    )
    def k(if_h, ifr_h, out_h, if2d, ones_v, iv2, ones2_v, zb, acc, sem):
        c = lax.axis_index("c")
        s = lax.axis_index("s")
        wid = s * _NC + c
        one16 = jnp.full((16,), 1.0, jnp.float32)
        zero16 = jnp.zeros((16,), jnp.float32)

        def fill_z(i, _):
            zb[i, :] = zero16
            return 0
        lax.fori_loop(0, _CPT_STEP, fill_z, 0)

        def fill_o(i, _):
            ones_v[i, :] = one16
            return 0
        lax.fori_loop(0, _K, fill_o, 0)
        for i in range(_REM):
            ones2_v[i, :] = one16

        pltpu.sync_copy(zb, acc.at[pl.ds(s * _CPT_STEP, _CPT_STEP)])

        @pl.when(s == _NS - 1)
        def _():
            pltpu.sync_copy(zb.at[pl.ds(0, _CPT_TAIL)],
                            acc.at[pl.ds(_NS * _CPT_STEP, _CPT_TAIL)])

        pltpu.sync_copy(if_h.at[wid], if2d)
        pltpu.sync_copy(ifr_h.at[wid], iv2)
        plsc.subcore_barrier()

        def chunk(jj, _):
            cps = [pltpu.async_copy(ones_v, acc.at[if2d.at[jj * 6 + t]],
                                    sem, add=True)
                   for t in range(6)]
            for cp in cps:
                cp.wait()
            return 0
        lax.fori_loop(0, _NFULL // 6, chunk, 0)
        pltpu.sync_copy(ones2_v, acc.at[iv2], add=True)

        plsc.subcore_barrier()
        dsto = pl.multiple_of(c * (R * N) + s * _CPT_STEP, 8)
        pltpu.sync_copy(acc.at[pl.ds(s * _CPT_STEP, _CPT_STEP)],
                        out_h.at[pl.ds(dsto, _CPT_STEP)])

        @pl.when(s == _NS - 1)
        def _():
            to = pl.multiple_of(c * (R * N) + _NS * _CPT_STEP, 8)
            pltpu.sync_copy(acc.at[pl.ds(_NS * _CPT_STEP, _CPT_TAIL)],
                            out_h.at[pl.ds(to, _CPT_TAIL)])

    return k(if3, if_r)


def _sc_edge(m_tbl, f_tbl, zeros_nh, im3, if3, dv3, im_r, if_r, dv_r):
    mesh = plsc.VectorSubcoreMesh(
        core_axis_name="c", subcore_axis_name="s",
        num_cores=_NC, num_subcores=_NS)

    @functools.partial(
        pl.kernel,
        out_type=jax.ShapeDtypeStruct((_NC * N, HID), jnp.float32),
        mesh=mesh,
        compiler_params=pltpu.CompilerParams(
            use_tc_tiling_on_sc=False, needs_layout_passes=False),
        scratch_types=[
            pltpu.VMEM((_PH0, _K), jnp.int32),
            pltpu.VMEM((_PH0, _K), jnp.int32),
            pltpu.VMEM((_PH0, _K), jnp.int32),
            pltpu.VMEM((_K, HID), jnp.float32),
            pltpu.VMEM((_K, 2 * HID), jnp.float32),
            pltpu.VMEM((_K, HID), jnp.float32),
            pltpu.VMEM((_K, 2 * HID), jnp.float32),
            pltpu.VMEM((_K, HID), jnp.float32),
            pltpu.VMEM((_K, HID), jnp.float32),
            pltpu.VMEM((_REM,), jnp.int32),
            pltpu.VMEM((_REM,), jnp.int32),
            pltpu.VMEM((_REM,), jnp.int32),
            pltpu.SemaphoreType.DMA,
            pltpu.SemaphoreType.DMA,
            pltpu.SemaphoreType.DMA,
            pltpu.SemaphoreType.DMA,
            pltpu.SemaphoreType.DMA,
            pltpu.SemaphoreType.DMA,
            pltpu.VMEM_SHARED((N, HID), jnp.float32),
        ],
    )
    def k(mtbl, ftbl, z_h, im_h, if_h, dv_h, imr_h, ifr_h, dvr_h, out_h,
          im2d, if2d, dv2d, mb0, fb0, mb1, fb1, mg0, mg1,
          imr, ifr, dvr,
          sm0, sf0, sm1, sf1, ss0, ss1, acc):
        c = lax.axis_index("c")
        s = lax.axis_index("s")
        wid = s * _NC + c

        pltpu.sync_copy(z_h.at[pl.ds(s * _RPT_STEP, _RPT_STEP)],
                        acc.at[pl.ds(s * _RPT_STEP, _RPT_STEP)])

        @pl.when(s == _NS - 1)
        def _():
            pltpu.sync_copy(z_h.at[pl.ds(_NS * _RPT_STEP, _RPT_TAIL)],
                            acc.at[pl.ds(_NS * _RPT_STEP, _RPT_TAIL)])

        pltpu.sync_copy(imr_h.at[wid], imr)
        pltpu.sync_copy(ifr_h.at[wid], ifr)
        pltpu.sync_copy(dvr_h.at[wid], dvr)
        plsc.subcore_barrier()

        bufs = ((mb0, fb0, sm0, sf0), (mb1, fb1, sm1, sf1))
        mgs = ((mg0, ss0), (mg1, ss1))

        def g_issue(j, slot):
            mb, fb, smx, sfx = bufs[slot]
            pltpu.async_copy(mtbl.at[im2d.at[j]], mb, smx)
            pltpu.async_copy(ftbl.at[if2d.at[j]], fb, sfx)

        def g_wait(j, slot):
            mb, fb, smx, sfx = bufs[slot]
            pltpu.make_async_copy(mtbl.at[im2d.at[j]], mb, smx).wait()
            pltpu.make_async_copy(ftbl.at[if2d.at[j]], fb, sfx).wait()

        def compute(mb, fb, mgx, nk, unroll):
            def edge(kk, _):
                for g in range(HID // 16):
                    mv = mb[kk, pl.ds(g * 16, 16)]
                    bv = fb[kk, pl.ds(g * 16, 16)]
                    gv = fb[kk, pl.ds(HID + g * 16, 16)]
                    v = gv * mv + bv
                    mgx[kk, pl.ds(g * 16, 16)] = jnp.maximum(v, 0.01 * v)
                return 0
            lax.fori_loop(0, nk, edge, 0, unroll=unroll)

        def run_phase(row0, nc):
            pltpu.sync_copy(im_h.at[wid, pl.ds(row0, nc)],
                            im2d.at[pl.ds(0, nc)])
            pltpu.sync_copy(if_h.at[wid, pl.ds(row0, nc)],
                            if2d.at[pl.ds(0, nc)])
            pltpu.sync_copy(dv_h.at[wid, pl.ds(row0, nc)],
                            dv2d.at[pl.ds(0, nc)])
            g_issue(0, 0)
            g_issue(1, 1)

            def chunk2(jj, _):
                j0 = jj * 2
                for slot in range(2):
                    j = j0 + slot
                    mb, fb, _, _ = bufs[slot]
                    mgx, ssx = mgs[slot]
                    g_wait(j, slot)

                    @pl.when(jj > 0)
                    def _():
                        pltpu.make_async_copy(
                            mgx, acc.at[dv2d.at[j]], ssx).wait()
                    compute(mb, fb, mgx, _K, 4)

                    @pl.when(j + 2 < nc)
                    def _():
                        g_issue(j + 2, slot)
                    pltpu.async_copy(mgx, acc.at[dv2d.at[j]], ssx, add=True)
                return 0
            lax.fori_loop(0, nc // 2, chunk2, 0)
            pltpu.make_async_copy(mg0, acc.at[dv2d.at[nc - 2]], ss0).wait()
            pltpu.make_async_copy(mg1, acc.at[dv2d.at[nc - 1]], ss1).wait()

        run_phase(0, _PH0)
        run_phase(_PH0, _PH1)

        cm = pltpu.async_copy(mtbl.at[imr], mb0.at[pl.ds(0, _REM)], sm0)
        cf = pltpu.async_copy(ftbl.at[ifr], fb0.at[pl.ds(0, _REM)], sf0)
        cm.wait()
        cf.wait()
        compute(mb0, fb0, mg0, _REM, 2)
        pltpu.sync_copy(mg0.at[pl.ds(0, _REM)], acc.at[dvr], add=True)

        plsc.subcore_barrier()
        dsto = pl.multiple_of(c * N + s * _RPT_STEP, 8)
        pltpu.sync_copy(acc.at[pl.ds(s * _RPT_STEP, _RPT_STEP)],
                        out_h.at[pl.ds(dsto, _RPT_STEP)])

        @pl.when(s == _NS - 1)
        def _():
            to = pl.multiple_of(c * N + _NS * _RPT_STEP, 8)
            pltpu.sync_copy(acc.at[pl.ds(_NS * _RPT_STEP, _RPT_TAIL)],
                            out_h.at[pl.ds(to, _RPT_TAIL)])

    return k(m_tbl, f_tbl, zeros_nh, im3, if3, dv3, im_r, if_r, dv_r)


def _rep(shape):
    nd = len(shape)
    return pl.BlockSpec(shape, lambda i: (0,) * nd)


def _tc_layer0(x, wcat, bcat, wtbl):
    def body(x_ref, wcat_ref, bcat_ref, wtbl_ref, sk_ref, m_ref, f_ref):
        hn = x_ref[...]
        big = jnp.dot(hn, wcat_ref[...],
                      preferred_element_type=jnp.float32) + bcat_ref[...]
        v = big[:, 2 * HID:3 * HID] * big[:, 0:HID] + big[:, HID:2 * HID]
        sk_ref[...] = _leaky(v)
        for r in range(R):
            m_ref[r] = big[:, 3 * HID + r * HID: 3 * HID + (r + 1) * HID]
            fv = big[:, 3 * HID + R * HID + r * 2 * HID:
                     3 * HID + R * HID + (r + 1) * 2 * HID]
            f_ref[r] = fv * wtbl_ref[:, r:r + 1]

    return pl.pallas_call(
        body,
        grid=(_GN,),
        in_specs=[
            pl.BlockSpec((_BN, D_IN), lambda i: (i, 0)),
            _rep((D_IN, _WCOLS)),
            _rep((1, _WCOLS)),
            pl.BlockSpec((_BN, R), lambda i: (i, 0)),
        ],
        out_specs=[
            pl.BlockSpec((_BN, HID), lambda i: (i, 0)),
            pl.BlockSpec((R, _BN, HID), lambda i: (0, i, 0)),
            pl.BlockSpec((R, _BN, 2 * HID), lambda i: (0, i, 0)),
        ],
        out_shape=[
            jax.ShapeDtypeStruct((N, HID), jnp.float32),
            jax.ShapeDtypeStruct((R, N, HID), jnp.float32),
            jax.ShapeDtypeStruct((R, N, 2 * HID), jnp.float32),
        ],
    )(x, wcat, bcat, wtbl)


def _tc_layer_hid(skip, p0, p1, lng, lnb, wcat, bcat, wtbl):
    def body(skip_ref, p0_ref, p1_ref, lng_ref, lnb_ref, wcat_ref, bcat_ref,
             wtbl_ref, h_ref, sk_ref, m_ref, f_ref):
        h = skip_ref[...] + p0_ref[...] + p1_ref[...]
        h_ref[...] = h
        mu = jnp.mean(h, axis=1, keepdims=True)
        d = h - mu
        var = jnp.mean(d * d, axis=1, keepdims=True)
        hn = d * lax.rsqrt(var + 1e-5) * lng_ref[...] + lnb_ref[...]
        big = jnp.dot(hn, wcat_ref[...],
                      preferred_element_type=jnp.float32) + bcat_ref[...]
        v = big[:, 2 * HID:3 * HID] * big[:, 0:HID] + big[:, HID:2 * HID]
        sk_ref[...] = _leaky(v)
        for r in range(R):
            m_ref[r] = big[:, 3 * HID + r * HID: 3 * HID + (r + 1) * HID]
            fv = big[:, 3 * HID + R * HID + r * 2 * HID:
                     3 * HID + R * HID + (r + 1) * 2 * HID]
            f_ref[r] = fv * wtbl_ref[:, r:r + 1]

    return pl.pallas_call(
        body,
        grid=(_GN,),
        in_specs=[
            pl.BlockSpec((_BN, HID), lambda i: (i, 0)),
            pl.BlockSpec((_BN, HID), lambda i: (i, 0)),
            pl.BlockSpec((_BN, HID), lambda i: (i, 0)),
            _rep((1, HID)),
            _rep((1, HID)),
            _rep((HID, _WCOLS)),
            _rep((1, _WCOLS)),
            pl.BlockSpec((_BN, R), lambda i: (i, 0)),
        ],
        out_specs=[
            pl.BlockSpec((_BN, HID), lambda i: (i, 0)),
            pl.BlockSpec((_BN, HID), lambda i: (i, 0)),
            pl.BlockSpec((R, _BN, HID), lambda i: (0, i, 0)),
            pl.BlockSpec((R, _BN, 2 * HID), lambda i: (0, i, 0)),
        ],
        out_shape=[
            jax.ShapeDtypeStruct((N, HID), jnp.float32),
            jax.ShapeDtypeStruct((N, HID), jnp.float32),
            jax.ShapeDtypeStruct((R, N, HID), jnp.float32),
            jax.ShapeDtypeStruct((R, N, 2 * HID), jnp.float32),
        ],
    )(skip, p0, p1, lng, lnb, wcat, bcat, wtbl)


def _tc_pool_a(precat, skip, p0, p1, batch3, sw1, sb1, sw2, sb2, sw3, sb3,
               tw1, tb1, tw2, tb2, tw3, tb3):
    def body(pc_ref, skip_ref, p0_ref, p1_ref, b_ref,
             sw1_ref, sb1_ref, sw2_ref, sb2_ref, sw3_ref, sb3_ref,
             tw1_ref, tb1_ref, tw2_ref, tb2_ref, tw3_ref, tb3_ref,
             cat_ref, sc_ref, tr_ref, smax_ref):
        h12 = skip_ref[...] + p0_ref[...] + p1_ref[...]
        cat = jnp.concatenate([pc_ref[...], h12], axis=1)
        cat_ref[...] = cat
        s1 = _leaky(jnp.dot(cat, sw1_ref[...],
                            preferred_element_type=jnp.float32) + sb1_ref[...])
        s2 = _leaky(jnp.dot(s1, sw2_ref[...],
                            preferred_element_type=jnp.float32) + sb2_ref[...])
        sc = jnp.dot(s2, sw3_ref[...],
                     preferred_element_type=jnp.float32) + sb3_ref[...]
        sc_ref[...] = sc
        t1 = _leaky(jnp.dot(cat, tw1_ref[...],
                            preferred_element_type=jnp.float32) + tb1_ref[...])
        t2 = _leaky(jnp.dot(t1, tw2_ref[...],
                            preferred_element_type=jnp.float32) + tb2_ref[...])
        tr = _leaky(jnp.dot(t2, tw3_ref[...],
                            preferred_element_type=jnp.float32) + tb3_ref[...])
        tr_ref[...] = tr

        b = b_ref[0, 0, :]
        ids = lax.broadcasted_iota(jnp.int32, (_BN, NG), 1)
        mask = b[:, None] == ids

        @pl.when(pl.program_id(0) == 0)
        def _():
            smax_ref[...] = jnp.full((NG, HEADS), -1e30, jnp.float32)

        cols = []
        for h in range(HEADS):
            cur = jnp.where(mask, sc[:, h:h + 1], -1e30)
            cols.append(jnp.max(cur, axis=0, keepdims=True))
        upd = jnp.concatenate(cols, axis=0).T
        smax_ref[...] = jnp.maximum(smax_ref[...], upd)

    return pl.pallas_call(
        body,
        grid=(_GN,),
        in_specs=[
            pl.BlockSpec((_BN, CAT - HID), lambda i: (i, 0)),
            pl.BlockSpec((_BN, HID), lambda i: (i, 0)),
            pl.BlockSpec((_BN, HID), lambda i: (i, 0)),
            pl.BlockSpec((_BN, HID), lambda i: (i, 0)),
            pl.BlockSpec((1, 1, _BN), lambda i: (i, 0, 0)),
            _rep((CAT, MLP_H)), _rep((1, MLP_H)),
            _rep((MLP_H, MLP_H)), _rep((1, MLP_H)),
            _rep((MLP_H, HEADS)), _rep((1, HEADS)),
            _rep((CAT, MLP_H)), _rep((1, MLP_H)),
            _rep((MLP_H, MLP_H)), _rep((1, MLP_H)),
            _rep((MLP_H, GREP)), _rep((1, GREP)),
        ],
        out_specs=[
            pl.BlockSpec((_BN, CAT), lambda i: (i, 0)),
            pl.BlockSpec((_BN, HEADS), lambda i: (i, 0)),
            pl.BlockSpec((_BN, GREP), lambda i: (i, 0)),
            pl.BlockSpec((NG, HEADS), lambda i: (0, 0)),
        ],
        out_shape=[
            jax.ShapeDtypeStruct((N, CAT), jnp.float32),
            jax.ShapeDtypeStruct((N, HEADS), jnp.float32),
            jax.ShapeDtypeStruct((N, GREP), jnp.float32),
            jax.ShapeDtypeStruct((NG, HEADS), jnp.float32),
        ],
    )(precat, skip, p0, p1, batch3, sw1, sb1, sw2, sb2, sw3, sb3,
      tw1, tb1, tw2, tb2, tw3, tb3)


def _tc_pool_b(scores, batch3, smax):
    def body(sc_ref, b_ref, smax_ref, ex_ref, ssum_ref):
        b = b_ref[0, 0, :]
        ids = lax.broadcasted_iota(jnp.int32, (_BN, NG), 1)
        maskf = (b[:, None] == ids).astype(jnp.float32)
        idsT = lax.broadcasted_iota(jnp.int32, (NG, _BN), 0)
        maskTf = (b[None, :] == idsT).astype(jnp.float32)
        smax_node = jnp.dot(maskf, smax_ref[...],
                            preferred_element_type=jnp.float32)
        ex = jnp.exp(sc_ref[...] - smax_node)
        ex_ref[...] = ex

        @pl.when(pl.program_id(0) == 0)
        def _():
            ssum_ref[...] = jnp.zeros((NG, HEADS), jnp.float32)

        ssum_ref[...] += jnp.dot(maskTf, ex,
                                 preferred_element_type=jnp.float32)

    return pl.pallas_call(
        body,
        grid=(_GN,),
        in_specs=[
            pl.BlockSpec((_BN, HEADS), lambda i: (i, 0)),
            pl.BlockSpec((1, 1, _BN), lambda i: (i, 0, 0)),
            _rep((NG, HEADS)),
        ],
        out_specs=[
            pl.BlockSpec((_BN, HEADS), lambda i: (i, 0)),
            pl.BlockSpec((NG, HEADS), lambda i: (0, 0)),
        ],
        out_shape=[
            jax.ShapeDtypeStruct((N, HEADS), jnp.float32),
            jax.ShapeDtypeStruct((NG, HEADS), jnp.float32),
        ],
    )(scores, batch3, smax)


def _tc_pool_c(ex, trans, batch3, ssum):
    def body(ex_ref, tr_ref, b_ref, ssum_ref, gr_ref):
        b = b_ref[0, 0, :]
        ids = lax.broadcasted_iota(jnp.int32, (_BN, NG), 1)
        maskf = (b[:, None] == ids).astype(jnp.float32)
        idsT = lax.broadcasted_iota(jnp.int32, (NG, _BN), 0)
        maskTf = (b[None, :] == idsT).astype(jnp.float32)
        ssum_node = jnp.dot(maskf, ssum_ref[...],
                            preferred_element_type=jnp.float32)
        w = ex_ref[...] / jnp.maximum(ssum_node, 1e-16)
        hsel = (lax.broadcasted_iota(jnp.int32, (HEADS, GREP), 1)
                // (GREP // HEADS)
                == lax.broadcasted_iota(jnp.int32, (HEADS, GREP), 0)
                ).astype(jnp.float32)
        wrep = jnp.dot(w, hsel, preferred_element_type=jnp.float32)
        weighted = wrep * tr_ref[...]

        @pl.when(pl.program_id(0) == 0)
        def _():
            gr_ref[...] = jnp.zeros((NG, GREP), jnp.float32)

        gr_ref[...] += jnp.dot(maskTf, weighted,
                               preferred_element_type=jnp.float32)

    return pl.pallas_call(
        body,
        grid=(_GN,),
        in_specs=[
            pl.BlockSpec((_BN, HEADS), lambda i: (i, 0)),
            pl.BlockSpec((_BN, GREP), lambda i: (i, 0)),
            pl.BlockSpec((1, 1, _BN), lambda i: (i, 0, 0)),
            _rep((NG, HEADS)),
        ],
        out_specs=[pl.BlockSpec((NG, GREP), lambda i: (0, 0))],
        out_shape=[jax.ShapeDtypeStruct((NG, GREP), jnp.float32)],
    )(ex, trans, batch3, ssum)[0]


def _cat_weights(lin_skip, film_skip_w, film_skip_b, lins, films_w, films_b):
    wcat = jnp.concatenate(
        [lin_skip, film_skip_w]
        + [lins[r] for r in range(R)]
        + [films_w[r] for r in range(R)], axis=1)
    bcat = jnp.concatenate(
        [jnp.zeros((HID,), jnp.float32), film_skip_b,
         jnp.zeros((R * HID,), jnp.float32), films_b.reshape(-1)])[None]
    return wcat, bcat


def kernel(x, edge_index, edge_type, batch, params):
    src = edge_index[0]
    dst = edge_index[1]
    et = edge_type.astype(jnp.int32)
    idx_m = et * N + src
    idx_f = et * N + dst

    def _split(a):
        a = a.reshape(_NW, _EPW)
        return (a[:, :_NFULL * _K].reshape(_NW, _NFULL, _K),
                a[:, _NFULL * _K:])

    im3, im_r = _split(idx_m)
    if3, if_r = _split(idx_f)
    dv3, dv_r = _split(dst)

    cnt2 = _sc_count(if3, if_r)
    cnt = cnt2.reshape(_NC, R * N, 16)[:, :, 0].sum(axis=0)
    wtbl = (1.0 / jnp.maximum(cnt, 1.0)).reshape(R, N).T

    p0 = params['l0']
    ph = params['hid']
    pp = params['pool']

    wcat0, bcat0 = _cat_weights(
        p0['lin_skip'], p0['film_skip_w'], p0['film_skip_b'],
        p0['lins'], p0['films_w'], p0['films_b'])
    zeros_nh = jnp.zeros((N, HID), jnp.float32)
    skip, m_all, f_all = _tc_layer0(x, wcat0, bcat0, wtbl)
    part = _sc_edge(m_all.reshape(R * N, HID), f_all.reshape(R * N, 2 * HID),
                    zeros_nh, im3, if3, dv3, im_r, if_r, dv_r
                    ).reshape(_NC, N, HID)

    hs = []
    for l in range(NUM_LAYERS):
        wcat, bcat = _cat_weights(
            ph['lin_skip'][l], ph['film_skip_w'][l], ph['film_skip_b'][l],
            ph['lins'][l], ph['films_w'][l], ph['films_b'][l])
        h_prev, skip, m_all, f_all = _tc_layer_hid(
            skip, part[0], part[1], ph['ln_g'][l][None], ph['ln_b'][l][None],
            wcat, bcat, wtbl)
        hs.append(h_prev)
        part = _sc_edge(
            m_all.reshape(R * N, HID), f_all.reshape(R * N, 2 * HID),
            zeros_nh, im3, if3, dv3, im_r, if_r, dv_r).reshape(_NC, N, HID)

    precat = jnp.concatenate(hs, axis=1)
    batch3 = batch.astype(jnp.int32).reshape(_GN, 1, _BN)
    cat, scores, trans, smax = _tc_pool_a(
        precat, skip, part[0], part[1], batch3,
        pp['score_w1'], pp['score_b1'][None], pp['score_w2'],
        pp['score_b2'][None], pp['score_w3'], pp['score_b3'][None],
        pp['trans_w1'], pp['trans_b1'][None], pp['trans_w2'],
        pp['trans_b2'][None], pp['trans_w3'], pp['trans_b3'][None])
    ex, ssum = _tc_pool_b(scores, batch3, smax)
    graph_reprs = _tc_pool_c(ex, trans, batch3, ssum)
    return graph_reprs, cat
